# baseline (speedup 1.0000x reference)
import functools

import jax
import jax.numpy as jnp
from jax import lax
from jax.experimental import pallas as pl
from jax.experimental.pallas import tpu as pltpu
from jax.experimental.pallas import tpu_sc as plsc

S = 7
C = 20
NBOX = 2
CELL = C + NBOX * 5
BATCH = 8
MID = 2048
IN_F = 1024 * S * S
OUT_F = S * S * CELL

K_TC = 33 * 1024
K_SC = IN_F - K_TC
K_BLK = 1024
KT_TILES = K_TC // K_BLK

NCORE = 2
NSUB = 16
NW = NCORE * NSUB
NCW = 16
COLS_PER_W = MID // NCW
KGRP = NW // NCW
K_PER_GRP = K_SC // KGRP
NV = 4
CHUNK = 512
N_CHUNKS = K_PER_GRP // CHUNK


def _stream_kernel(x_ref, w1_ref, b1_ref, h_ref):
    k = pl.program_id(0)

    @pl.when(k == 0)
    def _init():
        h_ref[...] = jnp.broadcast_to(b1_ref[...], h_ref.shape)

    h_ref[...] += jnp.dot(
        x_ref[...], w1_ref[...], preferred_element_type=jnp.float32
    )


_SC_MESH = plsc.VectorSubcoreMesh(core_axis_name="c", subcore_axis_name="s")


@functools.partial(
    pl.kernel,
    out_type=jax.ShapeDtypeStruct((KGRP, BATCH, MID), jnp.float32),
    mesh=_SC_MESH,
    scratch_types=[
        pltpu.VMEM((CHUNK, COLS_PER_W), jnp.float32),
        pltpu.VMEM((BATCH, CHUNK), jnp.float32),
        pltpu.VMEM((BATCH, COLS_PER_W), jnp.float32),
    ],
)
def _sc_partial(x_hbm, w1_hbm, out_hbm, wbuf, xbuf, hbuf):
    cid = lax.axis_index("c")
    sid = lax.axis_index("s")
    wid = cid * NSUB + sid
    grp = wid // NCW
    col0 = (wid % NCW) * COLS_PER_W
    kbase = K_TC + grp * K_PER_GRP

    for b in range(BATCH):
        for v in range(COLS_PER_W // 16):
            hbuf[b, pl.ds(v * 16, 16)] = jnp.zeros((16,), jnp.float32)

    def chunk_body(ci, carry):
        k0 = kbase + ci * CHUNK
        pltpu.sync_copy(
            w1_hbm.at[pl.ds(k0, CHUNK), pl.ds(col0, COLS_PER_W)], wbuf
        )
        pltpu.sync_copy(x_hbm.at[:, pl.ds(k0, CHUNK)], xbuf)

        for half in range(2):

            def group_body(g, a, _half=half):
                j0 = g * 16
                xvs = [xbuf[b, pl.ds(j0, 16)] for b in range(BATCH)]
                a = list(a)
                for l in range(16):
                    ws = [
                        wbuf[j0 + l, pl.ds(_half * 64 + v * 16, 16)]
                        for v in range(NV)
                    ]
                    for b in range(BATCH):
                        xs = xvs[b][l]
                        for v in range(NV):
                            a[b * NV + v] = a[b * NV + v] + xs * ws[v]
                return tuple(a)

            accs = tuple(
                jnp.zeros((16,), jnp.float32) for _ in range(BATCH * NV)
            )
            accs = lax.fori_loop(0, CHUNK // 16, group_body, accs)
            for b in range(BATCH):
                for v in range(NV):
                    sl = pl.ds(half * 64 + v * 16, 16)
                    hbuf[b, sl] = hbuf[b, sl] + accs[b * NV + v]
        return carry

    lax.fori_loop(0, N_CHUNKS, chunk_body, 0)
    pltpu.sync_copy(hbuf, out_hbm.at[grp, :, pl.ds(col0, COLS_PER_W)])


def _head_kernel(h1_ref, h2_ref, w2_ref, b2_ref, out_ref):
    h = h1_ref[...] + h2_ref[0] + h2_ref[1]
    h = jnp.where(h > 0, h, 0.1 * h)
    o = jnp.dot(h, w2_ref[...], preferred_element_type=jnp.float32)
    o = o + b2_ref[...]
    col = jax.lax.broadcasted_iota(jnp.int32, o.shape, 1)
    r = col % CELL
    m = (r < C) & ((r % 5 == 1) | (r % 5 == 2))
    out_ref[...] = jnp.where(m, jax.nn.sigmoid(o), o)


def kernel(x, W1, b1, W2, b2):
    x2 = x.reshape(BATCH, IN_F)
    h_sc = _sc_partial(x2, W1)
    h_tc = pl.pallas_call(
        _stream_kernel,
        grid=(KT_TILES,),
        in_specs=[
            pl.BlockSpec((BATCH, K_BLK), lambda k: (0, k)),
            pl.BlockSpec((K_BLK, MID), lambda k: (k, 0)),
            pl.BlockSpec((1, MID), lambda k: (0, 0)),
        ],
        out_specs=pl.BlockSpec((BATCH, MID), lambda k: (0, 0)),
        out_shape=jax.ShapeDtypeStruct((BATCH, MID), jnp.float32),
        compiler_params=pltpu.CompilerParams(
            dimension_semantics=("arbitrary",),
        ),
    )(x2, W1, b1[None, :])
    out = pl.pallas_call(
        _head_kernel,
        out_shape=jax.ShapeDtypeStruct((BATCH, OUT_F), jnp.float32),
    )(h_tc, h_sc, W2, b2[None, :])
    return out.reshape(-1, S, S, CELL)


# speedup vs baseline: 1.7485x; 1.7485x over previous
import functools

import jax
import jax.numpy as jnp
from jax import lax
from jax.experimental import pallas as pl
from jax.experimental.pallas import tpu as pltpu
from jax.experimental.pallas import tpu_sc as plsc

S = 7
C = 20
NBOX = 2
CELL = C + NBOX * 5
BATCH = 8
MID = 2048
IN_F = 1024 * S * S
OUT_F = S * S * CELL

K_TC = 42 * 1024
K_SC = IN_F - K_TC
K_BLK = 1024
KT_TILES = K_TC // K_BLK

NCORE = 2
NSUB = 16
NW = NCORE * NSUB
NCW = 16
COLS_PER_W = MID // NCW
KGRP = NW // NCW
K_PER_GRP = K_SC // KGRP
NV = 4
CHUNK = 512
N_CHUNKS = K_PER_GRP // CHUNK


def _stream_kernel(x_ref, w1_ref, b1_ref, h_ref):
    k = pl.program_id(0)

    @pl.when(k == 0)
    def _init():
        h_ref[...] = jnp.broadcast_to(b1_ref[...], h_ref.shape)

    h_ref[...] += jnp.dot(
        x_ref[...], w1_ref[...], preferred_element_type=jnp.float32
    )


_SC_MESH = plsc.VectorSubcoreMesh(core_axis_name="c", subcore_axis_name="s")


@functools.partial(
    pl.kernel,
    out_type=jax.ShapeDtypeStruct((KGRP, BATCH, MID), jnp.float32),
    mesh=_SC_MESH,
    scratch_types=[
        pltpu.VMEM((CHUNK, COLS_PER_W), jnp.float32),
        pltpu.VMEM((BATCH, CHUNK), jnp.float32),
        pltpu.VMEM((BATCH, COLS_PER_W), jnp.float32),
    ],
)
def _sc_partial(x_hbm, w1_hbm, out_hbm, wbuf, xbuf, hbuf):
    cid = lax.axis_index("c")
    sid = lax.axis_index("s")
    wid = cid * NSUB + sid
    grp = wid // NCW
    col0 = (wid % NCW) * COLS_PER_W
    kbase = K_TC + grp * K_PER_GRP

    for b in range(BATCH):
        for v in range(COLS_PER_W // 16):
            hbuf[b, pl.ds(v * 16, 16)] = jnp.zeros((16,), jnp.float32)

    def chunk_body(ci, carry):
        k0 = kbase + ci * CHUNK
        pltpu.sync_copy(
            w1_hbm.at[pl.ds(k0, CHUNK), pl.ds(col0, COLS_PER_W)], wbuf
        )
        pltpu.sync_copy(x_hbm.at[:, pl.ds(k0, CHUNK)], xbuf)

        for half in range(2):

            def group_body(g, a, _half=half):
                j0 = g * 16
                xvs = [xbuf[b, pl.ds(j0, 16)] for b in range(BATCH)]
                a = list(a)
                for l in range(16):
                    ws = [
                        wbuf[j0 + l, pl.ds(_half * 64 + v * 16, 16)]
                        for v in range(NV)
                    ]
                    for b in range(BATCH):
                        xs = xvs[b][l]
                        for v in range(NV):
                            a[b * NV + v] = a[b * NV + v] + xs * ws[v]
                return tuple(a)

            accs = tuple(
                jnp.zeros((16,), jnp.float32) for _ in range(BATCH * NV)
            )
            accs = lax.fori_loop(0, CHUNK // 16, group_body, accs)
            for b in range(BATCH):
                for v in range(NV):
                    sl = pl.ds(half * 64 + v * 16, 16)
                    hbuf[b, sl] = hbuf[b, sl] + accs[b * NV + v]
        return carry

    lax.fori_loop(0, N_CHUNKS, chunk_body, 0)
    pltpu.sync_copy(hbuf, out_hbm.at[grp, :, pl.ds(col0, COLS_PER_W)])


def _head_kernel(h1_ref, h2_ref, w2_ref, b2_ref, out_ref):
    h = h1_ref[...] + h2_ref[0] + h2_ref[1]
    h = jnp.where(h > 0, h, 0.1 * h)
    o = jnp.dot(h, w2_ref[...], preferred_element_type=jnp.float32)
    o = o + b2_ref[...]
    col = jax.lax.broadcasted_iota(jnp.int32, o.shape, 1)
    r = col % CELL
    m = (r < C) & ((r % 5 == 1) | (r % 5 == 2))
    out_ref[...] = jnp.where(m, jax.nn.sigmoid(o), o)


def kernel(x, W1, b1, W2, b2):
    x2 = x.reshape(BATCH, IN_F)
    h_sc = _sc_partial(x2, W1)
    h_tc = pl.pallas_call(
        _stream_kernel,
        grid=(KT_TILES,),
        in_specs=[
            pl.BlockSpec((BATCH, K_BLK), lambda k: (0, k)),
            pl.BlockSpec((K_BLK, MID), lambda k: (k, 0)),
            pl.BlockSpec((1, MID), lambda k: (0, 0)),
        ],
        out_specs=pl.BlockSpec((BATCH, MID), lambda k: (0, 0)),
        out_shape=jax.ShapeDtypeStruct((BATCH, MID), jnp.float32),
        compiler_params=pltpu.CompilerParams(
            dimension_semantics=("arbitrary",),
        ),
    )(x2, W1, b1[None, :])
    out = pl.pallas_call(
        _head_kernel,
        out_shape=jax.ShapeDtypeStruct((BATCH, OUT_F), jnp.float32),
    )(h_tc, h_sc, W2, b2[None, :])
    return out.reshape(-1, S, S, CELL)


# speedup vs baseline: 2.0690x; 1.1833x over previous
import jax
import jax.numpy as jnp
from jax.experimental import pallas as pl
from jax.experimental.pallas import tpu as pltpu

S = 7
C = 20
NBOX = 2
CELL = C + NBOX * 5
BATCH = 8
MID = 2048
IN_F = 1024 * S * S
OUT_F = S * S * CELL
K_BLK = 1024
K_TILES = IN_F // K_BLK


def _stream_kernel(x_ref, w1_ref, b1_ref, h_ref):
    k = pl.program_id(0)

    @pl.when(k == 0)
    def _init():
        h_ref[...] = jnp.broadcast_to(b1_ref[...], h_ref.shape)

    h_ref[...] += jnp.dot(
        x_ref[...], w1_ref[...], preferred_element_type=jnp.float32
    )


def _head_kernel(h_ref, w2_ref, b2_ref, out_ref):
    h = h_ref[...]
    h = jnp.where(h > 0, h, 0.1 * h)
    o = jnp.dot(h, w2_ref[...], preferred_element_type=jnp.float32)
    o = o + b2_ref[...]
    col = jax.lax.broadcasted_iota(jnp.int32, o.shape, 1)
    r = col % CELL
    m = (r < C) & ((r % 5 == 1) | (r % 5 == 2))
    out_ref[...] = jnp.where(m, jax.nn.sigmoid(o), o)


def kernel(x, W1, b1, W2, b2):
    x2 = x.reshape(BATCH, IN_F)
    h = pl.pallas_call(
        _stream_kernel,
        grid=(K_TILES,),
        in_specs=[
            pl.BlockSpec((BATCH, K_BLK), lambda k: (0, k)),
            pl.BlockSpec((K_BLK, MID), lambda k: (k, 0)),
            pl.BlockSpec((1, MID), lambda k: (0, 0)),
        ],
        out_specs=pl.BlockSpec((BATCH, MID), lambda k: (0, 0)),
        out_shape=jax.ShapeDtypeStruct((BATCH, MID), jnp.float32),
        compiler_params=pltpu.CompilerParams(
            dimension_semantics=("arbitrary",),
        ),
    )(x2, W1, b1[None, :])
    out = pl.pallas_call(
        _head_kernel,
        out_shape=jax.ShapeDtypeStruct((BATCH, OUT_F), jnp.float32),
    )(h, W2, b2[None, :])
    return out.reshape(-1, S, S, CELL)


# speedup vs baseline: 2.0736x; 1.0022x over previous
import jax
import jax.numpy as jnp
from jax.experimental import pallas as pl
from jax.experimental.pallas import tpu as pltpu

S = 7
C = 20
NBOX = 2
CELL = C + NBOX * 5
BATCH = 8
MID = 2048
IN_F = 1024 * S * S
OUT_F = S * S * CELL
K_BLK = 1024
K_TILES = IN_F // K_BLK


def _stream_kernel(x_ref, w1_ref, b1_ref, h_ref):
    k = pl.program_id(0)

    @pl.when(k == 0)
    def _init():
        h_ref[...] = jnp.broadcast_to(b1_ref[...], h_ref.shape)

    h_ref[...] += jnp.dot(
        x_ref[...], w1_ref[...], preferred_element_type=jnp.float32
    )


def _head_kernel(h_ref, w2_hbm, b2_ref, out_ref, w2_vmem, sem):
    cp = pltpu.make_async_copy(w2_hbm, w2_vmem, sem)
    cp.start()
    h = h_ref[...]
    h = jnp.where(h > 0, h, 0.1 * h)
    cp.wait()
    o = jnp.dot(h, w2_vmem[...], preferred_element_type=jnp.float32)
    o = o + b2_ref[...]
    col = jax.lax.broadcasted_iota(jnp.int32, o.shape, 1)
    r = col % CELL
    m = (r < C) & ((r % 5 == 1) | (r % 5 == 2))
    out_ref[...] = jnp.where(m, jax.nn.sigmoid(o), o)


def kernel(x, W1, b1, W2, b2):
    x2 = x.reshape(BATCH, IN_F)
    h = pl.pallas_call(
        _stream_kernel,
        grid=(K_TILES,),
        in_specs=[
            pl.BlockSpec((BATCH, K_BLK), lambda k: (0, k)),
            pl.BlockSpec((K_BLK, MID), lambda k: (k, 0)),
            pl.BlockSpec((1, MID), lambda k: (0, 0)),
        ],
        out_specs=pl.BlockSpec((BATCH, MID), lambda k: (0, 0)),
        out_shape=jax.ShapeDtypeStruct((BATCH, MID), jnp.float32),
        compiler_params=pltpu.CompilerParams(
            dimension_semantics=("arbitrary",),
        ),
    )(x2, W1, b1[None, :])
    out = pl.pallas_call(
        _head_kernel,
        in_specs=[
            pl.BlockSpec((BATCH, MID), lambda: (0, 0)),
            pl.BlockSpec(memory_space=pl.ANY),
            pl.BlockSpec((1, OUT_F), lambda: (0, 0)),
        ],
        out_specs=pl.BlockSpec((BATCH, OUT_F), lambda: (0, 0)),
        out_shape=jax.ShapeDtypeStruct((BATCH, OUT_F), jnp.float32),
        scratch_shapes=[
            pltpu.VMEM((MID, OUT_F), jnp.float32),
            pltpu.SemaphoreType.DMA,
        ],
    )(h, W2, b2[None, :])
    return out.reshape(-1, S, S, CELL)


# speedup vs baseline: 2.2167x; 1.0690x over previous
import jax
import jax.numpy as jnp
from jax.experimental import pallas as pl
from jax.experimental.pallas import tpu as pltpu

S = 7
C = 20
NBOX = 2
CELL = C + NBOX * 5
BATCH = 8
MID = 2048
IN_F = 1024 * S * S
OUT_F = S * S * CELL
K_BLK = 1024
K_TILES = IN_F // K_BLK


def _stream_kernel(x_ref, w1_ref, b1_ref, h_ref):
    k = pl.program_id(0)

    @pl.when(k == 0)
    def _init():
        h_ref[...] = jnp.broadcast_to(b1_ref[...], h_ref.shape)

    h_ref[...] += jnp.dot(
        x_ref[...], w1_ref[...], preferred_element_type=jnp.float32
    )


def _head_kernel(h_ref, w2t_ref, b2_ref, out_ref):
    h = h_ref[...]
    h = jnp.where(h > 0, h, 0.1 * h)
    o = jax.lax.dot_general(
        h, w2t_ref[...],
        dimension_numbers=(((1,), (1,)), ((), ())),
        preferred_element_type=jnp.float32,
    )
    o = o + b2_ref[...]
    col = jax.lax.broadcasted_iota(jnp.int32, o.shape, 1)
    r = col % CELL
    m = (r < C) & ((r % 5 == 1) | (r % 5 == 2))
    out_ref[...] = jnp.where(m, jax.nn.sigmoid(o), o)


def kernel(x, W1, b1, W2, b2):
    x2 = x.reshape(BATCH, IN_F)
    h = pl.pallas_call(
        _stream_kernel,
        grid=(K_TILES,),
        in_specs=[
            pl.BlockSpec((BATCH, K_BLK), lambda k: (0, k)),
            pl.BlockSpec((K_BLK, MID), lambda k: (k, 0)),
            pl.BlockSpec((1, MID), lambda k: (0, 0)),
        ],
        out_specs=pl.BlockSpec((BATCH, MID), lambda k: (0, 0)),
        out_shape=jax.ShapeDtypeStruct((BATCH, MID), jnp.float32),
        compiler_params=pltpu.CompilerParams(
            dimension_semantics=("arbitrary",),
        ),
    )(x2, W1, b1[None, :])
    out = pl.pallas_call(
        _head_kernel,
        out_shape=jax.ShapeDtypeStruct((BATCH, OUT_F), jnp.float32),
    )(h, W2.T, b2[None, :])
    return out.reshape(-1, S, S, CELL)
